# baseline (speedup 1.0000x reference)
import jax
import jax.numpy as jnp
from jax import lax
from jax.experimental import pallas as pl
from jax.experimental.pallas import tpu as pltpu

_LOG2PI = 1.8378770664093453
_LOG2E = 1.4426950408889634
_LN2 = 0.6931471805599453


def _gmm_body(z_ref, mu_ref, lv_ref, w_ref, out_ref,
              mu2_ref, p2_ref, c2_ref, cmax_ref, mub_ref, p2b_ref):
    @pl.when(pl.program_id(0) == 0)
    def _prep():
        lv = lv_ref[...]
        mu = mu_ref[...]
        wb = w_ref[...]
        wm = jnp.max(wb, keepdims=True)
        t = jnp.sum(jnp.exp(wb - wm), keepdims=True)
        logw = wb - wm - jnp.log(t * (2.0 / wb.shape[1]))
        p2 = (0.5 * _LOG2E) * jnp.exp(-lv)
        c2 = _LOG2E * ((-0.5 * _LOG2PI) - 0.5 * lv + logw)
        cmax = jnp.max(c2)
        G = lv.shape[0]
        mu2_ref[...] = jnp.broadcast_to(mu[:, None, :], (G, 8, 128))
        p2_ref[...] = jnp.broadcast_to(p2[:, None, :], (G, 8, 128))
        c2_ref[...] = jnp.broadcast_to((c2 - cmax)[:, None, :], (G, 8, 128))
        cmax_ref[...] = jnp.full((8, 128), cmax, jnp.float32)
        mub_ref[...] = jnp.broadcast_to(
            mu.astype(jnp.bfloat16)[:, None, :], (G, 8, 128))
        p2b_ref[...] = jnp.broadcast_to(
            p2.astype(jnp.bfloat16)[:, None, :], (G, 8, 128))

    zb = z_ref[...]
    zh = zb.reshape(zb.shape[0] // 8, 8, 64)
    z = jnp.concatenate([zh, zh], axis=-1)

    def q_of(g):
        d = z - mu2_ref[g][None]
        return p2_ref[g][None] * (d * d)

    zb16 = z.astype(jnp.bfloat16)

    def qb_of(g):
        d = zb16 - mub_ref[g][None]
        return p2b_ref[g][None] * (d * d)

    def pass1(g, mn):
        return jnp.minimum(mn, qb_of(g))

    mnb = lax.fori_loop(1, 32, pass1, qb_of(0), unroll=True)
    mn = mnb.astype(jnp.float32)
    mn = jnp.minimum(mn, pltpu.roll(mn, 64, 2))

    def pass2(g, s):
        return s + jnp.exp2(c2_ref[g][None] + (mn - q_of(g)))

    s0 = jnp.exp2(c2_ref[0][None] + (mn - q_of(0)))
    s = lax.fori_loop(1, 32, pass2, s0, unroll=True)
    s = s + pltpu.roll(s, 64, 2)
    res = _LN2 * (cmax_ref[0, 0] - mn) + jnp.log(s)
    out_ref[...] = res[..., :64].reshape(zb.shape[0], 64)


@jax.jit
def kernel(z, means, logvars, w):
    B, L = z.shape
    K = means.shape[0]
    mu2 = means.reshape(K // 2, 2 * L)
    lv2 = logvars.reshape(K // 2, 2 * L)
    wb = jnp.broadcast_to(w.reshape(K, 1), (K, L)).reshape(K // 2, 2 * L)

    RB = 64
    ROWS = 8 * RB
    grid = B // ROWS

    return pl.pallas_call(
        _gmm_body,
        grid=(grid,),
        in_specs=[
            pl.BlockSpec((ROWS, L), lambda i: (i, 0)),
            pl.BlockSpec((K // 2, 2 * L), lambda i: (0, 0)),
            pl.BlockSpec((K // 2, 2 * L), lambda i: (0, 0)),
            pl.BlockSpec((K // 2, 2 * L), lambda i: (0, 0)),
        ],
        out_specs=pl.BlockSpec((ROWS, L), lambda i: (i, 0)),
        out_shape=jax.ShapeDtypeStruct((B, L), jnp.float32),
        scratch_shapes=[
            pltpu.VMEM((K // 2, 8, 2 * L), jnp.float32),
            pltpu.VMEM((K // 2, 8, 2 * L), jnp.float32),
            pltpu.VMEM((K // 2, 8, 2 * L), jnp.float32),
            pltpu.VMEM((8, 2 * L), jnp.float32),
            pltpu.VMEM((K // 2, 8, 2 * L), jnp.bfloat16),
            pltpu.VMEM((K // 2, 8, 2 * L), jnp.bfloat16),
        ],
    )(z, mu2, lv2, wb)


# speedup vs baseline: 1.1840x; 1.1840x over previous
import jax
import jax.numpy as jnp
from jax import lax
from jax.experimental import pallas as pl
from jax.experimental.pallas import tpu as pltpu

_LOG2PI = 1.8378770664093453
_LOG2E = 1.4426950408889634
_LN2 = 0.6931471805599453


def _gmm_body(z_ref, mu_ref, lv_ref, w_ref, out_ref,
              mu2_ref, p2_ref, c2_ref, cmax_ref):
    @pl.when(pl.program_id(0) == 0)
    def _prep():
        lv = lv_ref[...]
        mu = mu_ref[...]
        wb = w_ref[...]
        wm = jnp.max(wb, keepdims=True)
        t = jnp.sum(jnp.exp(wb - wm), keepdims=True)
        logw = wb - wm - jnp.log(t * (2.0 / wb.shape[1]))
        p2 = (0.5 * _LOG2E) * jnp.exp(-lv)
        c2 = _LOG2E * ((-0.5 * _LOG2PI) - 0.5 * lv + logw)
        cmax = jnp.max(c2)
        G = lv.shape[0]
        mu2_ref[...] = jnp.broadcast_to(mu[:, None, :], (G, 8, 128))
        p2_ref[...] = jnp.broadcast_to(p2[:, None, :], (G, 8, 128))
        c2_ref[...] = jnp.broadcast_to((c2 - cmax)[:, None, :], (G, 8, 128))
        cmax_ref[...] = jnp.full((8, 128), cmax, jnp.float32)

    zb = z_ref[...]
    zh = zb.reshape(zb.shape[0] // 8, 8, 64)
    z = jnp.concatenate([zh, zh], axis=-1)

    def q_of(g):
        d = z - mu2_ref[g][None]
        return p2_ref[g][None] * (d * d)

    def pass1(g, mn):
        return jnp.minimum(mn, q_of(g))

    mn = lax.fori_loop(1, 32, pass1, q_of(0), unroll=True)
    mn = jnp.minimum(mn, pltpu.roll(mn, 64, 2))

    def pass2(g, s):
        return s + jnp.exp2(c2_ref[g][None] + (mn - q_of(g)))

    s0 = jnp.exp2(c2_ref[0][None] + (mn - q_of(0)))
    s = lax.fori_loop(1, 32, pass2, s0, unroll=True)
    s = s + pltpu.roll(s, 64, 2)
    res = _LN2 * (cmax_ref[0, 0] - mn) + jnp.log(s)
    out_ref[...] = res[..., :64].reshape(zb.shape[0], 64)


@jax.jit
def kernel(z, means, logvars, w):
    B, L = z.shape
    K = means.shape[0]
    mu2 = means.reshape(K // 2, 2 * L)
    lv2 = logvars.reshape(K // 2, 2 * L)
    wb = jnp.broadcast_to(w.reshape(K, 1), (K, L)).reshape(K // 2, 2 * L)

    RB = 64
    ROWS = 8 * RB
    grid = B // ROWS

    return pl.pallas_call(
        _gmm_body,
        grid=(grid,),
        in_specs=[
            pl.BlockSpec((ROWS, L), lambda i: (i, 0)),
            pl.BlockSpec((K // 2, 2 * L), lambda i: (0, 0)),
            pl.BlockSpec((K // 2, 2 * L), lambda i: (0, 0)),
            pl.BlockSpec((K // 2, 2 * L), lambda i: (0, 0)),
        ],
        out_specs=pl.BlockSpec((ROWS, L), lambda i: (i, 0)),
        out_shape=jax.ShapeDtypeStruct((B, L), jnp.float32),
        scratch_shapes=[
            pltpu.VMEM((K // 2, 8, 2 * L), jnp.float32),
            pltpu.VMEM((K // 2, 8, 2 * L), jnp.float32),
            pltpu.VMEM((K // 2, 8, 2 * L), jnp.float32),
            pltpu.VMEM((8, 2 * L), jnp.float32),
        ],
    )(z, mu2, lv2, wb)
